# Initial kernel scaffold; baseline (speedup 1.0000x reference)
import jax
import jax.numpy as jnp
from jax.experimental import pallas as pl


def kernel(x, c1_w1, c1_b1, c1_g, c1_be, c1_w2, c1_b2, c2_w1, c2_b1, c2_g, c2_be, c2_w2, c2_b2, lp_w1, lp_b1, lp_g, lp_be, lp_w2, lp_b2):
    raise NotImplementedError("write your pallas kernel here")


# speedup vs baseline: 3.7762x; 3.7762x over previous
import functools

import jax
import jax.numpy as jnp
from jax.experimental import pallas as pl
from jax.experimental.pallas import tpu as pltpu

N = 4096
K = 16
R = 256
C = 512
NC = N // C


def _extract_topk(vals, idx, k):
    out_v, out_i = [], []
    big = jnp.int32(2147483647)
    for _ in range(k):
        m = jnp.min(vals, axis=1, keepdims=True)
        eq = vals == m
        cand = jnp.where(eq, idx, big)
        sel = jnp.min(cand, axis=1, keepdims=True)
        out_v.append(m)
        out_i.append(sel)
        vals = jnp.where(idx == sel, jnp.inf, vals)
    return jnp.concatenate(out_v, axis=1), jnp.concatenate(out_i, axis=1)


def _knn_body(xr_ref, sqr_ref, xf_ref, sqf_ref, idx_ref):
    i = pl.program_id(0)
    xr = xr_ref[...]
    sqr = sqr_ref[...]
    row = i * R + jax.lax.broadcasted_iota(jnp.int32, (R, C), 0)
    cand_v, cand_i = [], []
    for c in range(NC):
        xc = xf_ref[pl.ds(c * C, C), :]
        sqc = sqf_ref[0, pl.ds(c * C, C)]
        dot = jax.lax.dot_general(xr, xc, (((1,), (1,)), ((), ())),
                                  preferred_element_type=jnp.float32)
        col = c * C + jax.lax.broadcasted_iota(jnp.int32, (R, C), 1)
        d2 = (sqr + sqc[None, :]) - 2.0 * dot
        d2 = jnp.where(col == row, jnp.inf, d2)
        v, ix = _extract_topk(d2, col, K)
        cand_v.append(v)
        cand_i.append(ix)
    vals = jnp.concatenate(cand_v, axis=1)
    idx = jnp.concatenate(cand_i, axis=1)
    _, topi = _extract_topk(vals, idx, K)
    idx_ref[...] = topi


@functools.partial(jax.jit, static_argnames=("interpret",))
def _knn_pallas(x, interpret=False):
    d = x.shape[1]
    sq = jnp.sum(x * x, axis=1)
    return pl.pallas_call(
        _knn_body,
        grid=(N // R,),
        in_specs=[
            pl.BlockSpec((R, d), lambda i: (i, 0)),
            pl.BlockSpec((R, 1), lambda i: (i, 0)),
            pl.BlockSpec((N, d), lambda i: (0, 0)),
            pl.BlockSpec((1, N), lambda i: (0, 0)),
        ],
        out_specs=pl.BlockSpec((R, K), lambda i: (i, 0)),
        out_shape=jax.ShapeDtypeStruct((N, K), jnp.int32),
        interpret=interpret,
    )(x, sq[:, None], x, sq[None, :])


def _mlp_k(h, w1, b1, g, be, w2, b2, sigmoid=False):
    h = h @ w1 + b1
    mu = jnp.mean(h, axis=0)
    var = jnp.var(h, axis=0)
    h = (h - mu) / jnp.sqrt(var + 1e-5) * g + be
    h = jnp.where(h >= 0, h, 0.1 * h)
    out = h @ w2 + b2
    if sigmoid:
        out = jax.nn.sigmoid(out)
    return out


def _edge_conv_k(x, k, params):
    idx = _knn_pallas(x)
    n, d = x.shape
    x_j = x[idx]
    x_i = jnp.broadcast_to(x[:, None, :], (n, k, d))
    e = jnp.concatenate([x_i, x_j - x_i], axis=-1).reshape(n * k, 2 * d)
    m = _mlp_k(e, *params)
    return jnp.max(m.reshape(n, k, -1), axis=1)


def kernel(x, c1_w1, c1_b1, c1_g, c1_be, c1_w2, c1_b2, c2_w1, c2_b1, c2_g, c2_be, c2_w2, c2_b2, lp_w1, lp_b1, lp_g, lp_be, lp_w2, lp_b2):
    h = _edge_conv_k(x, 16, (c1_w1, c1_b1, c1_g, c1_be, c1_w2, c1_b2))
    h = _edge_conv_k(h, 16, (c2_w1, c2_b1, c2_g, c2_be, c2_w2, c2_b2))
    idx = _knn_pallas(h)
    n = h.shape[0]
    src = idx.reshape(-1)
    dst = jnp.repeat(jnp.arange(n), 16)
    edge_index = jnp.stack([src, dst])
    ef = jnp.concatenate([h[src], h[dst]], axis=1)
    link_probs = _mlp_k(ef, lp_w1, lp_b1, lp_g, lp_be, lp_w2, lp_b2, sigmoid=True)
    return (link_probs, edge_index)


# speedup vs baseline: 4.2391x; 1.1226x over previous
import functools

import jax
import jax.numpy as jnp
from jax import lax
from jax.experimental import pallas as pl
from jax.experimental.pallas import tpu as pltpu
from jax.experimental.pallas import tpu_sc as plsc

N = 4096
K = 16
NH = 20
R = 256
E4 = R * K
C = 512
NB = N // R


def _extract_topk(vals, idx, k):
    out_v, out_i = [], []
    big = jnp.int32(2147483647)
    for _ in range(k):
        m = jnp.min(vals, axis=1, keepdims=True)
        cand = jnp.where(vals == m, idx, big)
        sel = jnp.min(cand, axis=1, keepdims=True)
        out_v.append(m)
        out_i.append(sel)
        vals = jnp.where(idx == sel, jnp.inf, vals)
    return jnp.concatenate(out_v, axis=1), jnp.concatenate(out_i, axis=1)


def _knn_body(xr_ref, sqr_ref, xf_ref, sqf_ref, idx_ref):
    i = pl.program_id(0)
    xr = xr_ref[...]
    sqr = sqr_ref[...]
    row = i * R + lax.broadcasted_iota(jnp.int32, (R, C), 0)
    cand_v, cand_i = [], []
    for c in range(N // C):
        xc = xf_ref[pl.ds(c * C, C), :]
        sqc = sqf_ref[0, pl.ds(c * C, C)]
        dot = lax.dot_general(xr, xc, (((1,), (1,)), ((), ())),
                              preferred_element_type=jnp.float32)
        col = c * C + lax.broadcasted_iota(jnp.int32, (R, C), 1)
        d2 = (sqr + sqc[None, :]) - 2.0 * dot
        d2 = jnp.where(col == row, jnp.inf, d2)
        v, ix = _extract_topk(d2, col, K)
        cand_v.append(v)
        cand_i.append(ix)
    vals = jnp.concatenate(cand_v, axis=1)
    idx = jnp.concatenate(cand_i, axis=1)
    _, topi = _extract_topk(vals, idx, K)
    idx_ref[...] = topi


@functools.partial(jax.jit, static_argnames=("interpret",))
def _knn_pallas(x, interpret=False):
    d = x.shape[1]
    sq = jnp.sum(x * x, axis=1)
    return pl.pallas_call(
        _knn_body,
        grid=(NB,),
        in_specs=[
            pl.BlockSpec((R, d), lambda i: (i, 0)),
            pl.BlockSpec((R, 1), lambda i: (i, 0)),
            pl.BlockSpec((N, d), lambda i: (0, 0)),
            pl.BlockSpec((1, N), lambda i: (0, 0)),
        ],
        out_specs=pl.BlockSpec((R, K), lambda i: (i, 0)),
        out_shape=jax.ShapeDtypeStruct((N, K), jnp.int32),
        interpret=interpret,
    )(x, sq[:, None], x, sq[None, :])


def _sc_gather(table, idx):
    t_rows, d = table.shape
    b = idx.shape[0]
    info = plsc.get_sparse_core_info()
    ncores, nsub = info.num_cores, info.num_subcores
    nw = ncores * nsub
    b_per_w = b // nw
    ch = min(b_per_w, 128, max(8, (128 * 1024) // (d * 4)))
    while b_per_w % ch:
        ch //= 2
    mesh = plsc.VectorSubcoreMesh(core_axis_name="c", subcore_axis_name="s")

    @functools.partial(
        pl.kernel,
        out_type=jax.ShapeDtypeStruct((b, d), jnp.float32),
        mesh=mesh,
        scratch_types=[
            pltpu.VMEM((ch,), jnp.int32),
            pltpu.VMEM((ch, d), jnp.float32),
            pltpu.SemaphoreType.DMA,
        ],
    )
    def gk(table_hbm, idx_hbm, out_hbm, idx_v, rows_v, sem):
        wid = lax.axis_index("s") * ncores + lax.axis_index("c")
        base = wid * b_per_w
        for t in range(b_per_w // ch):
            off = base + t * ch
            pltpu.sync_copy(idx_hbm.at[pl.ds(off, ch)], idx_v)
            pltpu.async_copy(table_hbm.at[idx_v], rows_v, sem).wait()
            pltpu.sync_copy(rows_v, out_hbm.at[pl.ds(off, ch)])

    return gk(table, idx)


def _edge_pre_body(xr_ref, xj_ref, w1_ref, b1_ref, pre_ref, s_ref):
    d = xr_ref.shape[1]
    w1 = w1_ref[...]
    b1 = b1_ref[...]
    acc_s = jnp.zeros((1, NH), jnp.float32)
    for t in range(8):
        x32 = xr_ref[pl.ds(t * 32, 32), :]
        xi = jnp.broadcast_to(x32[:, None, :], (32, K, d)).reshape(32 * K, d)
        xj = xj_ref[pl.ds(t * 512, 512), :][:, :d]
        e = jnp.concatenate([xi, xj - xi], axis=1)
        pre = lax.dot_general(e, w1, (((1,), (0,)), ((), ())),
                              preferred_element_type=jnp.float32) + b1
        pre_ref[pl.ds(t * 512, 512), :] = pre
        acc_s = acc_s + jnp.sum(pre, axis=0, keepdims=True)
    s_ref[...] = acc_s[None]


def _bn_var_body(pre_ref, s_ref, q_ref):
    mu = jnp.sum(s_ref[...], axis=0) / float(N * K)
    dv = pre_ref[...] - mu
    q_ref[...] = jnp.sum(dv * dv, axis=0, keepdims=True)[None]


@functools.partial(jax.jit, static_argnames=("interpret",))
def _bn_var(pre, s, interpret=False):
    return pl.pallas_call(
        _bn_var_body,
        grid=(NB,),
        in_specs=[
            pl.BlockSpec((E4, NH), lambda i: (i, 0)),
            pl.BlockSpec((NB, 1, NH), lambda i: (0, 0, 0)),
        ],
        out_specs=pl.BlockSpec((1, 1, NH), lambda i: (i, 0, 0)),
        out_shape=jax.ShapeDtypeStruct((NB, 1, NH), jnp.float32),
        interpret=interpret,
    )(pre, s)


@functools.partial(jax.jit, static_argnames=("interpret",))
def _edge_pre(x, xj, w1, b1, interpret=False):
    d = x.shape[1]
    dj = xj.shape[1]
    return pl.pallas_call(
        _edge_pre_body,
        grid=(NB,),
        in_specs=[
            pl.BlockSpec((R, d), lambda i: (i, 0)),
            pl.BlockSpec((E4, dj), lambda i: (i, 0)),
            pl.BlockSpec((2 * d, NH), lambda i: (0, 0)),
            pl.BlockSpec((1, NH), lambda i: (0, 0)),
        ],
        out_specs=[
            pl.BlockSpec((E4, NH), lambda i: (i, 0)),
            pl.BlockSpec((1, 1, NH), lambda i: (i, 0, 0)),
        ],
        out_shape=[
            jax.ShapeDtypeStruct((N * K, NH), jnp.float32),
            jax.ShapeDtypeStruct((NB, 1, NH), jnp.float32),
        ],
        interpret=interpret,
    )(x, xj, w1, b1[None, :])


def _fin2_body(pre_ref, mu_ref, var_ref, g_ref, be_ref, w2_ref, b2_ref,
               out_ref, *, sigmoid):
    mu = mu_ref[...]
    var = var_ref[...]
    pre = pre_ref[...]
    hn = (pre - mu) / jnp.sqrt(var + 1e-5) * g_ref[...] + be_ref[...]
    hl = jnp.where(hn >= 0, hn, 0.1 * hn)
    m = lax.dot_general(hl, w2_ref[...], (((1,), (0,)), ((), ())),
                        preferred_element_type=jnp.float32) + b2_ref[...]
    if sigmoid:
        out_ref[...] = jax.nn.sigmoid(m)
    else:
        out_ref[...] = jnp.max(m.reshape(R, K, m.shape[1]), axis=1)


@functools.partial(jax.jit, static_argnames=("interpret",))
def _edge_fin2(pre, mu, var, g, be, w2, b2, interpret=False):
    h2 = w2.shape[1]
    return pl.pallas_call(
        functools.partial(_fin2_body, sigmoid=False),
        grid=(NB,),
        in_specs=[
            pl.BlockSpec((E4, NH), lambda i: (i, 0)),
            pl.BlockSpec((1, NH), lambda i: (0, 0)),
            pl.BlockSpec((1, NH), lambda i: (0, 0)),
            pl.BlockSpec((1, NH), lambda i: (0, 0)),
            pl.BlockSpec((1, NH), lambda i: (0, 0)),
            pl.BlockSpec((NH, h2), lambda i: (0, 0)),
            pl.BlockSpec((1, h2), lambda i: (0, 0)),
        ],
        out_specs=pl.BlockSpec((R, h2), lambda i: (i, 0)),
        out_shape=jax.ShapeDtypeStruct((N, h2), jnp.float32),
        interpret=interpret,
    )(pre, mu[None, :], var[None, :], g[None, :], be[None, :], w2, b2[None, :])


def _fin_body(pre_ref, s_ref, q_ref, g_ref, be_ref, w2_ref, b2_ref, out_ref,
              *, sigmoid):
    mu = jnp.sum(s_ref[...], axis=0) / float(N * K)
    var = jnp.sum(q_ref[...], axis=0) / float(N * K)
    pre = pre_ref[...]
    hn = (pre - mu) / jnp.sqrt(var + 1e-5) * g_ref[...] + be_ref[...]
    hl = jnp.where(hn >= 0, hn, 0.1 * hn)
    m = lax.dot_general(hl, w2_ref[...], (((1,), (0,)), ((), ())),
                        preferred_element_type=jnp.float32) + b2_ref[...]
    if sigmoid:
        out_ref[...] = jax.nn.sigmoid(m)
    else:
        out_ref[...] = jnp.max(m.reshape(R, K, m.shape[1]), axis=1)


@functools.partial(jax.jit, static_argnames=("interpret",))
def _edge_fin(pre, s, q, g, be, w2, b2, interpret=False):
    h2 = w2.shape[1]
    return pl.pallas_call(
        functools.partial(_fin_body, sigmoid=False),
        grid=(NB,),
        in_specs=[
            pl.BlockSpec((E4, NH), lambda i: (i, 0)),
            pl.BlockSpec((NB, 1, NH), lambda i: (0, 0, 0)),
            pl.BlockSpec((NB, 1, NH), lambda i: (0, 0, 0)),
            pl.BlockSpec((1, NH), lambda i: (0, 0)),
            pl.BlockSpec((1, NH), lambda i: (0, 0)),
            pl.BlockSpec((NH, h2), lambda i: (0, 0)),
            pl.BlockSpec((1, h2), lambda i: (0, 0)),
        ],
        out_specs=pl.BlockSpec((R, h2), lambda i: (i, 0)),
        out_shape=jax.ShapeDtypeStruct((N, h2), jnp.float32),
        interpret=interpret,
    )(pre, s, q, g[None, :], be[None, :], w2, b2[None, :])


@functools.partial(jax.jit, static_argnames=("interpret",))
def _link_fin(pre, s, q, g, be, w2, b2, interpret=False):
    return pl.pallas_call(
        functools.partial(_fin_body, sigmoid=True),
        grid=(NB,),
        in_specs=[
            pl.BlockSpec((E4, NH), lambda i: (i, 0)),
            pl.BlockSpec((NB, 1, NH), lambda i: (0, 0, 0)),
            pl.BlockSpec((NB, 1, NH), lambda i: (0, 0, 0)),
            pl.BlockSpec((1, NH), lambda i: (0, 0)),
            pl.BlockSpec((1, NH), lambda i: (0, 0)),
            pl.BlockSpec((NH, 1), lambda i: (0, 0)),
            pl.BlockSpec((1, 1), lambda i: (0, 0)),
        ],
        out_specs=pl.BlockSpec((E4, 1), lambda i: (i, 0)),
        out_shape=jax.ShapeDtypeStruct((N * K, 1), jnp.float32),
        interpret=interpret,
    )(pre, s, q, g[None, :], be[None, :], w2, b2[None, :])


def _pq_body(h_ref, w1_ref, p_ref, q_ref):
    h = h_ref[...]
    d = h.shape[1]
    w = w1_ref[...]
    p = lax.dot_general(h, w[:d], (((1,), (0,)), ((), ())),
                        preferred_element_type=jnp.float32)
    p_ref[...] = jnp.concatenate(
        [p, jnp.zeros((p.shape[0], 128 - NH), jnp.float32)], axis=1)
    q_ref[...] = lax.dot_general(h, w[d:], (((1,), (0,)), ((), ())),
                                 preferred_element_type=jnp.float32)


@functools.partial(jax.jit, static_argnames=("interpret",))
def _pq(h, w1, interpret=False):
    d = h.shape[1]
    return pl.pallas_call(
        _pq_body,
        grid=(1,),
        in_specs=[
            pl.BlockSpec((N, d), lambda i: (0, 0)),
            pl.BlockSpec((2 * d, NH), lambda i: (0, 0)),
        ],
        out_specs=[
            pl.BlockSpec((N, 128), lambda i: (0, 0)),
            pl.BlockSpec((N, NH), lambda i: (0, 0)),
        ],
        out_shape=[
            jax.ShapeDtypeStruct((N, 128), jnp.float32),
            jax.ShapeDtypeStruct((N, NH), jnp.float32),
        ],
        interpret=interpret,
    )(h, w1)


def _link_pre_body(pg_ref, q_ref, b1_ref, pre_ref, s_ref):
    pg = pg_ref[...][:, :NH]
    qq = q_ref[...]
    qe = jnp.broadcast_to(qq[:, None, :], (R, K, NH)).reshape(E4, NH)
    pre = pg + qe + b1_ref[...]
    pre_ref[...] = pre
    s_ref[...] = jnp.sum(pre, axis=0, keepdims=True)[None]


@functools.partial(jax.jit, static_argnames=("interpret",))
def _link_pre(pg, qv, b1, interpret=False):
    return pl.pallas_call(
        _link_pre_body,
        grid=(NB,),
        in_specs=[
            pl.BlockSpec((E4, 128), lambda i: (i, 0)),
            pl.BlockSpec((R, NH), lambda i: (i, 0)),
            pl.BlockSpec((1, NH), lambda i: (0, 0)),
        ],
        out_specs=[
            pl.BlockSpec((E4, NH), lambda i: (i, 0)),
            pl.BlockSpec((1, 1, NH), lambda i: (i, 0, 0)),
        ],
        out_shape=[
            jax.ShapeDtypeStruct((N * K, NH), jnp.float32),
            jax.ShapeDtypeStruct((NB, 1, NH), jnp.float32),
        ],
        interpret=interpret,
    )(pg, qv, b1[None, :])


def _bn_mlp_xla(h, w1, b1, g, be, w2, b2):
    h = h @ w1 + b1
    mu = jnp.mean(h, axis=0)
    var = jnp.var(h, axis=0)
    h = (h - mu) / jnp.sqrt(var + 1e-5) * g + be
    h = jnp.where(h >= 0, h, 0.1 * h)
    return h @ w2 + b2


def _edge_conv_xla(x, idx, params):
    w1, b1, g, be, w2, b2 = params
    n, d = x.shape
    x_j = x[idx]
    x_i = jnp.broadcast_to(x[:, None, :], (n, K, d))
    e = jnp.concatenate([x_i, x_j - x_i], axis=-1).reshape(n * K, 2 * d)
    m = _bn_mlp_xla(e, w1, b1, g, be, w2, b2)
    return jnp.max(m.reshape(n, K, -1), axis=1)


def kernel(x, c1_w1, c1_b1, c1_g, c1_be, c1_w2, c1_b2, c2_w1, c2_b1, c2_g, c2_be, c2_w2, c2_b2, lp_w1, lp_b1, lp_g, lp_be, lp_w2, lp_b2):
    idx1 = _knn_pallas(x)
    h1 = _edge_conv_xla(x, idx1, (c1_w1, c1_b1, c1_g, c1_be, c1_w2, c1_b2))
    idx2 = _knn_pallas(h1)
    h2 = _edge_conv_xla(h1, idx2, (c2_w1, c2_b1, c2_g, c2_be, c2_w2, c2_b2))

    idx3 = _knn_pallas(h2)
    src = idx3.reshape(-1)
    ppad, qv = _pq(h2, lp_w1)
    pg = _sc_gather(ppad, src)
    pre3, s3 = _link_pre(pg, qv, lp_b1)
    q3 = _bn_var(pre3, s3)
    link_probs = _link_fin(pre3, s3, q3, lp_g, lp_be, lp_w2, lp_b2)

    dst = jnp.repeat(jnp.arange(N), K)
    edge_index = jnp.stack([src, dst])
    return (link_probs, edge_index)
